# Initial kernel scaffold; baseline (speedup 1.0000x reference)
import jax
import jax.numpy as jnp
from jax.experimental import pallas as pl


def kernel(x, edge_index, edge_attr, W1, b1, W2, b2, W3, b3):
    raise NotImplementedError("write your pallas kernel here")


# speedup vs baseline: 9.3030x; 9.3030x over previous
import functools

import jax
import jax.numpy as jnp
from jax import lax
from jax.experimental import pallas as pl
from jax.experimental.pallas import tpu as pltpu
from jax.experimental.pallas import tpu_sc as plsc

_N = 10000
_D = 128
_E = 320000
_NC = 2
_NS = 16
_L = 16
_NW = _NC * _NS
_CH = 128
_CPW = 80
_EP = _NW * _CPW * _CH
_NP = 10240
_RPT = _NP // _NS

_mesh = plsc.VectorSubcoreMesh(
    core_axis_name="c", subcore_axis_name="s",
    num_cores=_NC, num_subcores=_NS)


@functools.partial(
    pl.kernel,
    compiler_params=pltpu.CompilerParams(needs_layout_passes=False),
    out_type=jax.ShapeDtypeStruct((_NC, _N), jnp.float32),
    mesh=_mesh,
    scratch_types=[
        pltpu.VMEM((_CPW, _CH), jnp.int32),
        pltpu.VMEM((_CPW, _CH), jnp.int32),
        pltpu.VMEM((_CPW, _CH), jnp.float32),
        pltpu.VMEM((_CPW, _CH), jnp.float32),
        pltpu.VMEM((_N,), jnp.float32),
        pltpu.VMEM((_N,), jnp.float32),
        pltpu.VMEM_SHARED((_N,), jnp.float32),
    ],
)
def _agg_narrow(y_hbm, src_hbm, dst_hbm, ew_hbm, out_hbm,
                src_v, dst_v, ew_v, msg_v, y_v, stage_v, acc_sh):
    c = lax.axis_index("c")
    s = lax.axis_index("s")
    w = c * _NS + s
    pltpu.sync_copy(src_hbm.at[w], src_v)
    pltpu.sync_copy(dst_hbm.at[w], dst_v)
    pltpu.sync_copy(ew_hbm.at[w], ew_v)
    pltpu.sync_copy(y_hbm, y_v)

    @pl.when(s == 0)
    def _():
        def zero_body(i, carry):
            stage_v[pl.ds(i * _L, _L)] = jnp.zeros((_L,), jnp.float32)
            return carry
        lax.fori_loop(0, _N // _L, zero_body, 0)
        pltpu.sync_copy(stage_v, acc_sh)

    plsc.subcore_barrier()

    def chunk_body(j, carry):
        for g in range(_CH // _L):
            sl = pl.ds(g * _L, _L)
            vals = plsc.load_gather(y_v, [src_v[j, sl]])
            msg_v[j, sl] = vals * ew_v[j, sl]
        pltpu.sync_copy(msg_v.at[j], acc_sh.at[dst_v.at[j]], add=True)
        return carry
    lax.fori_loop(0, _CPW, chunk_body, 0)

    plsc.subcore_barrier()

    @pl.when(s == 0)
    def _():
        pltpu.sync_copy(acc_sh, stage_v)
        pltpu.sync_copy(stage_v, out_hbm.at[c])


@functools.partial(
    pl.kernel,
    compiler_params=pltpu.CompilerParams(needs_layout_passes=False),
    out_type=jax.ShapeDtypeStruct((_NC, _NP, _D), jnp.float32),
    mesh=_mesh,
    scratch_types=[
        pltpu.VMEM((_CPW, _CH), jnp.int32),
        pltpu.VMEM((_CPW, _CH), jnp.int32),
        pltpu.VMEM((_CPW, _CH), jnp.float32),
        pltpu.VMEM((_CH, _D), jnp.float32),
        pltpu.VMEM_SHARED((_NP, _D), jnp.float32),
        pltpu.SemaphoreType.DMA,
    ],
)
def _agg_wide(y_hbm, src_hbm, dst_hbm, ew_hbm, out_hbm,
              src_v, dst_v, ew_v, rows_v, acc_sh, sem):
    c = lax.axis_index("c")
    s = lax.axis_index("s")
    w = c * _NS + s
    pltpu.sync_copy(src_hbm.at[w], src_v)
    pltpu.sync_copy(dst_hbm.at[w], dst_v)
    pltpu.sync_copy(ew_hbm.at[w], ew_v)

    def zero_body(i, carry):
        for f in range(_D // _L):
            rows_v[i, pl.ds(f * _L, _L)] = jnp.zeros((_L,), jnp.float32)
        return carry
    lax.fori_loop(0, _CH, zero_body, 0)
    for p in range(_RPT // _CH):
        pltpu.sync_copy(rows_v, acc_sh.at[pl.ds(s * _RPT + p * _CH, _CH)])

    plsc.subcore_barrier()

    def chunk_body(j, carry):
        pltpu.async_copy(y_hbm.at[src_v.at[j]], rows_v, sem).wait()
        jv = jnp.full((_L,), j, jnp.int32)

        def edge_body(e, c2):
            scale = plsc.load_gather(ew_v, [jv, jnp.full((_L,), e, jnp.int32)])
            for f in range(_D // _L):
                sl = pl.ds(f * _L, _L)
                rows_v[e, sl] = rows_v[e, sl] * scale
            return c2
        lax.fori_loop(0, _CH, edge_body, 0)
        pltpu.sync_copy(rows_v, acc_sh.at[dst_v.at[j]], add=True)
        return carry
    lax.fori_loop(0, _CPW, chunk_body, 0)

    plsc.subcore_barrier()

    for p in range(_RPT // _CH):
        sl = pl.ds(s * _RPT + p * _CH, _CH)
        pltpu.sync_copy(acc_sh.at[sl], rows_v)
        pltpu.sync_copy(rows_v, out_hbm.at[c, sl])


_R = 1000
_G = _N // _R


def _row_spec():
    return pl.BlockSpec((_R, _D), lambda i: (i, 0))


def _col_spec():
    return pl.BlockSpec((_R, 1), lambda i: (i, 0))


def _dense1_body(d0_ref, d1_ref, x_ref, w_ref, dis_ref, y_ref):
    deg = d0_ref[...] + d1_ref[...] + 1.0
    dis = lax.rsqrt(deg)
    xw = jnp.dot(x_ref[...], w_ref[...], preferred_element_type=jnp.float32)
    dis_ref[...] = dis
    y_ref[...] = dis * xw


def _dense1(d0, d1, x, W):
    return pl.pallas_call(
        _dense1_body,
        grid=(_G,),
        in_specs=[_col_spec(), _col_spec(), _row_spec(),
                  pl.BlockSpec((_D, _D), lambda i: (0, 0))],
        out_specs=[_col_spec(), _row_spec()],
        out_shape=[jax.ShapeDtypeStruct((_N, 1), jnp.float32),
                   jax.ShapeDtypeStruct((_N, _D), jnp.float32)],
    )(d0, d1, x, W)


def _dense_mid_body(p0_ref, p1_ref, y_ref, dis_ref, w_ref, b_ref, yn_ref):
    dis = dis_ref[...]
    h = jnp.maximum(
        dis * (p0_ref[...] + p1_ref[...] + y_ref[...]) + b_ref[...], 0.0)
    yn_ref[...] = dis * jnp.dot(h, w_ref[...],
                                preferred_element_type=jnp.float32)


def _dense_mid(p0, p1, y, dis, W, b):
    return pl.pallas_call(
        _dense_mid_body,
        grid=(_G,),
        in_specs=[_row_spec(), _row_spec(), _row_spec(), _col_spec(),
                  pl.BlockSpec((_D, _D), lambda i: (0, 0)),
                  pl.BlockSpec((1, _D), lambda i: (0, 0))],
        out_specs=_row_spec(),
        out_shape=jax.ShapeDtypeStruct((_N, _D), jnp.float32),
    )(p0, p1, y, dis, W, b)


def _dense3_body(p0_ref, p1_ref, y_ref, dis_ref, w3_ref, b_ref, y3_ref):
    dis = dis_ref[...]
    h = jnp.maximum(
        dis * (p0_ref[...] + p1_ref[...] + y_ref[...]) + b_ref[...], 0.0)
    y3_ref[...] = dis * jnp.sum(h * w3_ref[...], axis=1, keepdims=True)


def _dense3(p0, p1, y, dis, w3row, b):
    return pl.pallas_call(
        _dense3_body,
        grid=(_G,),
        in_specs=[_row_spec(), _row_spec(), _row_spec(), _col_spec(),
                  pl.BlockSpec((1, _D), lambda i: (0, 0)),
                  pl.BlockSpec((1, _D), lambda i: (0, 0))],
        out_specs=_col_spec(),
        out_shape=jax.ShapeDtypeStruct((_N, 1), jnp.float32),
    )(p0, p1, y, dis, w3row, b)


def _final_body(q0_ref, q1_ref, y3_ref, dis_ref, b3_ref, o_ref):
    o_ref[...] = (dis_ref[...] * (q0_ref[...] + q1_ref[...] + y3_ref[...])
                  + b3_ref[...])


def _final(q0, q1, y3, dis, b3):
    return pl.pallas_call(
        _final_body,
        grid=(_G,),
        in_specs=[_col_spec(), _col_spec(), _col_spec(), _col_spec(),
                  pl.BlockSpec((1, 1), lambda i: (0, 0))],
        out_specs=_col_spec(),
        out_shape=jax.ShapeDtypeStruct((_N, 1), jnp.float32),
    )(q0, q1, y3, dis, b3)


def kernel(x, edge_index, edge_attr, W1, b1, W2, b2, W3, b3):
    ei = edge_index.astype(jnp.int32)
    src, dst = ei[0], ei[1]
    ew = edge_attr[:, 0].astype(jnp.float32)

    pad = _EP - _E
    srcp = jnp.concatenate([src, jnp.zeros((pad,), jnp.int32)])
    dstp = jnp.concatenate([dst, jnp.zeros((pad,), jnp.int32)])
    ewp = jnp.concatenate([ew, jnp.zeros((pad,), jnp.float32)])
    srcp = srcp.reshape(_NW, _CPW, _CH)
    dstp = dstp.reshape(_NW, _CPW, _CH)
    ewp = ewp.reshape(_NW, _CPW, _CH)

    ones_n = jnp.ones((_N,), jnp.float32)
    degp = _agg_narrow(ones_n, srcp, dstp, ewp)
    dis, y1 = _dense1(degp[0][:, None], degp[1][:, None], x, W1)

    p1 = _agg_wide(y1, srcp, dstp, ewp)
    y2 = _dense_mid(p1[0, :_N], p1[1, :_N], y1, dis, W2, b1.reshape(1, _D))

    p2 = _agg_wide(y2, srcp, dstp, ewp)
    y3 = _dense3(p2[0, :_N], p2[1, :_N], y2, dis, W3.reshape(1, _D), b2.reshape(1, _D))

    q = _agg_narrow(y3[:, 0], srcp, dstp, ewp)
    out = _final(q[0][:, None], q[1][:, None], y3, dis, b3.reshape(1, 1))
    return out[:, 0]


# speedup vs baseline: 11.8243x; 1.2710x over previous
import functools

import jax
import jax.numpy as jnp
from jax import lax
from jax.experimental import pallas as pl
from jax.experimental.pallas import tpu as pltpu
from jax.experimental.pallas import tpu_sc as plsc

_N = 10000
_D = 128
_E = 320000
_NC = 2
_NS = 16
_L = 16
_NW = _NC * _NS
_CH = 128
_CPW = 80
_EP = _NW * _CPW * _CH
_NP = 10240
_RPT = _NP // _NS

_mesh = plsc.VectorSubcoreMesh(
    core_axis_name="c", subcore_axis_name="s",
    num_cores=_NC, num_subcores=_NS)


@functools.partial(
    pl.kernel,
    compiler_params=pltpu.CompilerParams(needs_layout_passes=False),
    out_type=jax.ShapeDtypeStruct((_NC, _N), jnp.float32),
    mesh=_mesh,
    scratch_types=[
        pltpu.VMEM((_CPW, _CH), jnp.int32),
        pltpu.VMEM((_CPW, _CH), jnp.int32),
        pltpu.VMEM((_CPW, _CH), jnp.float32),
        pltpu.VMEM((_CPW, _CH), jnp.float32),
        pltpu.VMEM((_N,), jnp.float32),
        pltpu.VMEM((_N,), jnp.float32),
        pltpu.VMEM_SHARED((_N,), jnp.float32),
    ],
)
def _agg_narrow(y_hbm, src_hbm, dst_hbm, ew_hbm, out_hbm,
                src_v, dst_v, ew_v, msg_v, y_v, stage_v, acc_sh):
    c = lax.axis_index("c")
    s = lax.axis_index("s")
    w = c * _NS + s
    pltpu.sync_copy(src_hbm.at[w], src_v)
    pltpu.sync_copy(dst_hbm.at[w], dst_v)
    pltpu.sync_copy(ew_hbm.at[w], ew_v)
    pltpu.sync_copy(y_hbm, y_v)

    @pl.when(s == 0)
    def _():
        def zero_body(i, carry):
            stage_v[pl.ds(i * _L, _L)] = jnp.zeros((_L,), jnp.float32)
            return carry
        lax.fori_loop(0, _N // _L, zero_body, 0)
        pltpu.sync_copy(stage_v, acc_sh)

    plsc.subcore_barrier()

    def chunk_body(j, carry):
        for g in range(_CH // _L):
            sl = pl.ds(g * _L, _L)
            vals = plsc.load_gather(y_v, [src_v[j, sl]])
            msg_v[j, sl] = vals * ew_v[j, sl]
        pltpu.sync_copy(msg_v.at[j], acc_sh.at[dst_v.at[j]], add=True)
        return carry
    lax.fori_loop(0, _CPW, chunk_body, 0)

    plsc.subcore_barrier()

    @pl.when(s == 0)
    def _():
        pltpu.sync_copy(acc_sh, stage_v)
        pltpu.sync_copy(stage_v, out_hbm.at[c])


_SCH = 64
_NSUB = _CH // _SCH
_TCH = _CPW * _NSUB
_NB = 2
_OUTER = _TCH // _NB


@functools.partial(
    pl.kernel,
    compiler_params=pltpu.CompilerParams(needs_layout_passes=False),
    out_type=jax.ShapeDtypeStruct((_NC, _NP, _D), jnp.float32),
    mesh=_mesh,
    scratch_types=[
        pltpu.VMEM((_CPW, _CH), jnp.int32),
        pltpu.VMEM((_CPW, _CH), jnp.int32),
        pltpu.VMEM((_CPW, _CH), jnp.float32),
        pltpu.VMEM((_SCH, _D), jnp.float32),
        pltpu.VMEM((_SCH, _D), jnp.float32),
        pltpu.VMEM_SHARED((_NP, _D), jnp.float32),
        pltpu.SemaphoreType.DMA,
        pltpu.SemaphoreType.DMA,
    ],
)
def _agg_wide(y_hbm, src_hbm, dst_hbm, ew_hbm, out_hbm,
              src_v, dst_v, ew_v, rows0, rows1,
              acc_sh, gs0, gs1):
    bufs = (rows0, rows1)
    sems = (gs0, gs1)
    c = lax.axis_index("c")
    s = lax.axis_index("s")
    w = c * _NS + s
    pltpu.sync_copy(src_hbm.at[w], src_v)
    pltpu.sync_copy(dst_hbm.at[w], dst_v)
    pltpu.sync_copy(ew_hbm.at[w], ew_v)

    def zero_body(i, carry):
        for f in range(_D // _L):
            rows0[i, pl.ds(f * _L, _L)] = jnp.zeros((_L,), jnp.float32)
        return carry
    lax.fori_loop(0, _SCH, zero_body, 0)
    for p in range(_RPT // _SCH):
        pltpu.sync_copy(rows0, acc_sh.at[pl.ds(s * _RPT + p * _SCH, _SCH)])

    plsc.subcore_barrier()

    def _gather_descr(t, buf, sem):
        j = t // _NSUB
        o = (t % _NSUB) * _SCH
        return pltpu.make_async_copy(
            y_hbm.at[src_v.at[j, pl.ds(o, _SCH)]], buf, sem)

    for b in range(_NB):
        _gather_descr(b, bufs[b], sems[b]).start()

    def outer_body(i, carry):
        for b in range(_NB):
            t = i * _NB + b
            j = t // _NSUB
            o = (t % _NSUB) * _SCH
            buf = bufs[b]
            _gather_descr(t, buf, sems[b]).wait()
            jv = jnp.full((_L,), j, jnp.int32)
            ov = jnp.full((_L,), o, jnp.int32)

            def edge_body(e, c2, buf=buf, jv=jv, ov=ov):
                scale = plsc.load_gather(
                    ew_v, [jv, ov + jnp.full((_L,), e, jnp.int32)])
                for f in range(_D // _L):
                    sl = pl.ds(f * _L, _L)
                    buf[e, sl] = buf[e, sl] * scale
                return c2
            lax.fori_loop(0, _SCH, edge_body, 0)
            pltpu.sync_copy(
                buf, acc_sh.at[dst_v.at[j, pl.ds(o, _SCH)]], add=True)

            @pl.when(i < _OUTER - 1)
            def _(b=b, t=t, buf=buf):
                _gather_descr(t + _NB, buf, sems[b]).start()
        return carry
    lax.fori_loop(0, _OUTER, outer_body, 0)

    plsc.subcore_barrier()

    for p in range(_RPT // _SCH):
        sl = pl.ds(s * _RPT + p * _SCH, _SCH)
        pltpu.sync_copy(acc_sh.at[sl], rows0)
        pltpu.sync_copy(rows0, out_hbm.at[c, sl])


_R = 1000
_G = _N // _R


def _row_spec():
    return pl.BlockSpec((_R, _D), lambda i: (i, 0))


def _col_spec():
    return pl.BlockSpec((_R, 1), lambda i: (i, 0))


def _dense1_body(d0_ref, d1_ref, x_ref, w_ref, dis_ref, y_ref):
    deg = d0_ref[...] + d1_ref[...] + 1.0
    dis = lax.rsqrt(deg)
    xw = jnp.dot(x_ref[...], w_ref[...], preferred_element_type=jnp.float32)
    dis_ref[...] = dis
    y_ref[...] = dis * xw


def _dense1(d0, d1, x, W):
    return pl.pallas_call(
        _dense1_body,
        grid=(_G,),
        in_specs=[_col_spec(), _col_spec(), _row_spec(),
                  pl.BlockSpec((_D, _D), lambda i: (0, 0))],
        out_specs=[_col_spec(), _row_spec()],
        out_shape=[jax.ShapeDtypeStruct((_N, 1), jnp.float32),
                   jax.ShapeDtypeStruct((_N, _D), jnp.float32)],
    )(d0, d1, x, W)


def _dense_mid_body(p0_ref, p1_ref, y_ref, dis_ref, w_ref, b_ref, yn_ref):
    dis = dis_ref[...]
    h = jnp.maximum(
        dis * (p0_ref[...] + p1_ref[...] + y_ref[...]) + b_ref[...], 0.0)
    yn_ref[...] = dis * jnp.dot(h, w_ref[...],
                                preferred_element_type=jnp.float32)


def _dense_mid(p0, p1, y, dis, W, b):
    return pl.pallas_call(
        _dense_mid_body,
        grid=(_G,),
        in_specs=[_row_spec(), _row_spec(), _row_spec(), _col_spec(),
                  pl.BlockSpec((_D, _D), lambda i: (0, 0)),
                  pl.BlockSpec((1, _D), lambda i: (0, 0))],
        out_specs=_row_spec(),
        out_shape=jax.ShapeDtypeStruct((_N, _D), jnp.float32),
    )(p0, p1, y, dis, W, b)


def _dense3_body(p0_ref, p1_ref, y_ref, dis_ref, w3_ref, b_ref, y3_ref):
    dis = dis_ref[...]
    h = jnp.maximum(
        dis * (p0_ref[...] + p1_ref[...] + y_ref[...]) + b_ref[...], 0.0)
    y3_ref[...] = dis * jnp.sum(h * w3_ref[...], axis=1, keepdims=True)


def _dense3(p0, p1, y, dis, w3row, b):
    return pl.pallas_call(
        _dense3_body,
        grid=(_G,),
        in_specs=[_row_spec(), _row_spec(), _row_spec(), _col_spec(),
                  pl.BlockSpec((1, _D), lambda i: (0, 0)),
                  pl.BlockSpec((1, _D), lambda i: (0, 0))],
        out_specs=_col_spec(),
        out_shape=jax.ShapeDtypeStruct((_N, 1), jnp.float32),
    )(p0, p1, y, dis, w3row, b)


def _final_body(q0_ref, q1_ref, y3_ref, dis_ref, b3_ref, o_ref):
    o_ref[...] = (dis_ref[...] * (q0_ref[...] + q1_ref[...] + y3_ref[...])
                  + b3_ref[...])


def _final(q0, q1, y3, dis, b3):
    return pl.pallas_call(
        _final_body,
        grid=(_G,),
        in_specs=[_col_spec(), _col_spec(), _col_spec(), _col_spec(),
                  pl.BlockSpec((1, 1), lambda i: (0, 0))],
        out_specs=_col_spec(),
        out_shape=jax.ShapeDtypeStruct((_N, 1), jnp.float32),
    )(q0, q1, y3, dis, b3)


def kernel(x, edge_index, edge_attr, W1, b1, W2, b2, W3, b3):
    ei = edge_index.astype(jnp.int32)
    src, dst = ei[0], ei[1]
    ew = edge_attr[:, 0].astype(jnp.float32)

    pad = _EP - _E
    srcp = jnp.concatenate([src, jnp.zeros((pad,), jnp.int32)])
    dstp = jnp.concatenate([dst, jnp.zeros((pad,), jnp.int32)])
    ewp = jnp.concatenate([ew, jnp.zeros((pad,), jnp.float32)])
    srcp = srcp.reshape(_NW, _CPW, _CH)
    dstp = dstp.reshape(_NW, _CPW, _CH)
    ewp = ewp.reshape(_NW, _CPW, _CH)

    ones_n = jnp.ones((_N,), jnp.float32)
    degp = _agg_narrow(ones_n, srcp, dstp, ewp)
    dis, y1 = _dense1(degp[0][:, None], degp[1][:, None], x, W1)

    p1 = _agg_wide(y1, srcp, dstp, ewp)
    y2 = _dense_mid(p1[0, :_N], p1[1, :_N], y1, dis, W2, b1.reshape(1, _D))

    p2 = _agg_wide(y2, srcp, dstp, ewp)
    y3 = _dense3(p2[0, :_N], p2[1, :_N], y2, dis, W3.reshape(1, _D), b2.reshape(1, _D))

    q = _agg_narrow(y3[:, 0], srcp, dstp, ewp)
    out = _final(q[0][:, None], q[1][:, None], y3, dis, b3.reshape(1, 1))
    return out[:, 0]


# speedup vs baseline: 11.8652x; 1.0035x over previous
import functools

import jax
import jax.numpy as jnp
from jax import lax
from jax.experimental import pallas as pl
from jax.experimental.pallas import tpu as pltpu
from jax.experimental.pallas import tpu_sc as plsc

_N = 10000
_D = 128
_E = 320000
_NC = 2
_NS = 16
_L = 16
_NW = _NC * _NS
_CH = 128
_CPW = 80
_EP = _NW * _CPW * _CH
_NP = 10240
_RPT = _NP // _NS

_mesh = plsc.VectorSubcoreMesh(
    core_axis_name="c", subcore_axis_name="s",
    num_cores=_NC, num_subcores=_NS)


@functools.partial(
    pl.kernel,
    compiler_params=pltpu.CompilerParams(needs_layout_passes=False),
    out_type=jax.ShapeDtypeStruct((_NC, _N), jnp.float32),
    mesh=_mesh,
    scratch_types=[
        pltpu.VMEM((_CPW, _CH), jnp.int32),
        pltpu.VMEM((_CPW, _CH), jnp.int32),
        pltpu.VMEM((_CPW, _CH), jnp.float32),
        pltpu.VMEM((_CPW, _CH), jnp.float32),
        pltpu.VMEM((_N,), jnp.float32),
        pltpu.VMEM((_N,), jnp.float32),
        pltpu.VMEM_SHARED((_N,), jnp.float32),
    ],
)
def _agg_narrow(y_hbm, src_hbm, dst_hbm, ew_hbm, out_hbm,
                src_v, dst_v, ew_v, msg_v, y_v, stage_v, acc_sh):
    c = lax.axis_index("c")
    s = lax.axis_index("s")
    w = c * _NS + s
    pltpu.sync_copy(src_hbm.at[w], src_v)
    pltpu.sync_copy(dst_hbm.at[w], dst_v)
    pltpu.sync_copy(ew_hbm.at[w], ew_v)
    pltpu.sync_copy(y_hbm, y_v)

    @pl.when(s == 0)
    def _():
        def zero_body(i, carry):
            stage_v[pl.ds(i * _L, _L)] = jnp.zeros((_L,), jnp.float32)
            return carry
        lax.fori_loop(0, _N // _L, zero_body, 0)
        pltpu.sync_copy(stage_v, acc_sh)

    plsc.subcore_barrier()

    def chunk_body(j, carry):
        for g in range(_CH // _L):
            sl = pl.ds(g * _L, _L)
            vals = plsc.load_gather(y_v, [src_v[j, sl]])
            msg_v[j, sl] = vals * ew_v[j, sl]
        pltpu.sync_copy(msg_v.at[j], acc_sh.at[dst_v.at[j]], add=True)
        return carry
    lax.fori_loop(0, _CPW, chunk_body, 0)

    plsc.subcore_barrier()

    @pl.when(s == 0)
    def _():
        pltpu.sync_copy(acc_sh, out_hbm.at[c])


_SCH = 64
_NSUB = _CH // _SCH
_TCH = _CPW * _NSUB
_NB = 2
_OUTER = _TCH // _NB


@functools.partial(
    pl.kernel,
    compiler_params=pltpu.CompilerParams(needs_layout_passes=False),
    out_type=jax.ShapeDtypeStruct((_NC, _NP, _D), jnp.float32),
    mesh=_mesh,
    scratch_types=[
        pltpu.VMEM((_CPW, _CH), jnp.int32),
        pltpu.VMEM((_CPW, _CH), jnp.int32),
        pltpu.VMEM((_CPW, _CH), jnp.float32),
        pltpu.VMEM((_SCH, _D), jnp.float32),
        pltpu.VMEM((_SCH, _D), jnp.float32),
        pltpu.VMEM_SHARED((_NP, _D), jnp.float32),
        pltpu.SemaphoreType.DMA,
        pltpu.SemaphoreType.DMA,
    ],
)
def _agg_wide(y_hbm, src_hbm, dst_hbm, ew_hbm, out_hbm,
              src_v, dst_v, ew_v, rows0, rows1,
              acc_sh, gs0, gs1):
    bufs = (rows0, rows1)
    sems = (gs0, gs1)
    c = lax.axis_index("c")
    s = lax.axis_index("s")
    w = c * _NS + s
    pltpu.sync_copy(src_hbm.at[w], src_v)
    pltpu.sync_copy(dst_hbm.at[w], dst_v)
    pltpu.sync_copy(ew_hbm.at[w], ew_v)

    def zero_body(i, carry):
        for f in range(_D // _L):
            rows0[i, pl.ds(f * _L, _L)] = jnp.zeros((_L,), jnp.float32)
        return carry
    lax.fori_loop(0, _SCH, zero_body, 0)
    for p in range(_RPT // _SCH):
        pltpu.async_copy(
            rows0, acc_sh.at[pl.ds(s * _RPT + p * _SCH, _SCH)],
            sems[p % _NB])
    for p in range(_RPT // _SCH):
        pltpu.make_async_copy(
            rows0, acc_sh.at[pl.ds(s * _RPT + p * _SCH, _SCH)],
            sems[p % _NB]).wait()

    plsc.subcore_barrier()

    def _gather_descr(t, buf, sem):
        j = t // _NSUB
        o = (t % _NSUB) * _SCH
        return pltpu.make_async_copy(
            y_hbm.at[src_v.at[j, pl.ds(o, _SCH)]], buf, sem)

    for b in range(_NB):
        _gather_descr(b, bufs[b], sems[b]).start()

    def outer_body(i, carry):
        for b in range(_NB):
            t = i * _NB + b
            j = t // _NSUB
            o = (t % _NSUB) * _SCH
            buf = bufs[b]
            _gather_descr(t, buf, sems[b]).wait()
            jv = jnp.full((_L,), j, jnp.int32)
            ov = jnp.full((_L,), o, jnp.int32)

            def edge_body(e, c2, buf=buf, jv=jv, ov=ov):
                scale = plsc.load_gather(
                    ew_v, [jv, ov + jnp.full((_L,), e, jnp.int32)])
                for f in range(_D // _L):
                    sl = pl.ds(f * _L, _L)
                    buf[e, sl] = buf[e, sl] * scale
                return c2
            lax.fori_loop(0, _SCH, edge_body, 0)
            pltpu.sync_copy(
                buf, acc_sh.at[dst_v.at[j, pl.ds(o, _SCH)]], add=True)

            @pl.when(i < _OUTER - 1)
            def _(b=b, t=t, buf=buf):
                _gather_descr(t + _NB, buf, sems[b]).start()
        return carry
    lax.fori_loop(0, _OUTER, outer_body, 0)

    plsc.subcore_barrier()

    pltpu.sync_copy(acc_sh.at[pl.ds(s * _RPT, _RPT)],
                    out_hbm.at[c, pl.ds(s * _RPT, _RPT)])


_R = 1000
_G = _N // _R


def _row_spec():
    return pl.BlockSpec((_R, _D), lambda i: (i, 0))


def _col_spec():
    return pl.BlockSpec((_R, 1), lambda i: (i, 0))


def _dense1_body(d0_ref, d1_ref, x_ref, w_ref, dis_ref, y_ref):
    deg = d0_ref[...] + d1_ref[...] + 1.0
    dis = lax.rsqrt(deg)
    xw = jnp.dot(x_ref[...], w_ref[...], preferred_element_type=jnp.float32)
    dis_ref[...] = dis
    y_ref[...] = dis * xw


def _dense1(d0, d1, x, W):
    return pl.pallas_call(
        _dense1_body,
        grid=(_G,),
        in_specs=[_col_spec(), _col_spec(), _row_spec(),
                  pl.BlockSpec((_D, _D), lambda i: (0, 0))],
        out_specs=[_col_spec(), _row_spec()],
        out_shape=[jax.ShapeDtypeStruct((_N, 1), jnp.float32),
                   jax.ShapeDtypeStruct((_N, _D), jnp.float32)],
    )(d0, d1, x, W)


def _dense_mid_body(p0_ref, p1_ref, y_ref, dis_ref, w_ref, b_ref, yn_ref):
    dis = dis_ref[...]
    h = jnp.maximum(
        dis * (p0_ref[...] + p1_ref[...] + y_ref[...]) + b_ref[...], 0.0)
    yn_ref[...] = dis * jnp.dot(h, w_ref[...],
                                preferred_element_type=jnp.float32)


def _dense_mid(p0, p1, y, dis, W, b):
    return pl.pallas_call(
        _dense_mid_body,
        grid=(_G,),
        in_specs=[_row_spec(), _row_spec(), _row_spec(), _col_spec(),
                  pl.BlockSpec((_D, _D), lambda i: (0, 0)),
                  pl.BlockSpec((1, _D), lambda i: (0, 0))],
        out_specs=_row_spec(),
        out_shape=jax.ShapeDtypeStruct((_N, _D), jnp.float32),
    )(p0, p1, y, dis, W, b)


def _dense3_body(p0_ref, p1_ref, y_ref, dis_ref, w3_ref, b_ref, y3_ref):
    dis = dis_ref[...]
    h = jnp.maximum(
        dis * (p0_ref[...] + p1_ref[...] + y_ref[...]) + b_ref[...], 0.0)
    y3_ref[...] = dis * jnp.sum(h * w3_ref[...], axis=1, keepdims=True)


def _dense3(p0, p1, y, dis, w3row, b):
    return pl.pallas_call(
        _dense3_body,
        grid=(_G,),
        in_specs=[_row_spec(), _row_spec(), _row_spec(), _col_spec(),
                  pl.BlockSpec((1, _D), lambda i: (0, 0)),
                  pl.BlockSpec((1, _D), lambda i: (0, 0))],
        out_specs=_col_spec(),
        out_shape=jax.ShapeDtypeStruct((_N, 1), jnp.float32),
    )(p0, p1, y, dis, w3row, b)


def _final_body(q0_ref, q1_ref, y3_ref, dis_ref, b3_ref, o_ref):
    o_ref[...] = (dis_ref[...] * (q0_ref[...] + q1_ref[...] + y3_ref[...])
                  + b3_ref[...])


def _final(q0, q1, y3, dis, b3):
    return pl.pallas_call(
        _final_body,
        grid=(_G,),
        in_specs=[_col_spec(), _col_spec(), _col_spec(), _col_spec(),
                  pl.BlockSpec((1, 1), lambda i: (0, 0))],
        out_specs=_col_spec(),
        out_shape=jax.ShapeDtypeStruct((_N, 1), jnp.float32),
    )(q0, q1, y3, dis, b3)


def kernel(x, edge_index, edge_attr, W1, b1, W2, b2, W3, b3):
    ei = edge_index.astype(jnp.int32)
    src, dst = ei[0], ei[1]
    ew = edge_attr[:, 0].astype(jnp.float32)

    pad = _EP - _E
    srcp = jnp.concatenate([src, jnp.zeros((pad,), jnp.int32)])
    dstp = jnp.concatenate([dst, jnp.zeros((pad,), jnp.int32)])
    ewp = jnp.concatenate([ew, jnp.zeros((pad,), jnp.float32)])
    srcp = srcp.reshape(_NW, _CPW, _CH)
    dstp = dstp.reshape(_NW, _CPW, _CH)
    ewp = ewp.reshape(_NW, _CPW, _CH)

    ones_n = jnp.ones((_N,), jnp.float32)
    degp = _agg_narrow(ones_n, srcp, dstp, ewp)
    dis, y1 = _dense1(degp[0][:, None], degp[1][:, None], x, W1)

    p1 = _agg_wide(y1, srcp, dstp, ewp)
    y2 = _dense_mid(p1[0, :_N], p1[1, :_N], y1, dis, W2, b1.reshape(1, _D))

    p2 = _agg_wide(y2, srcp, dstp, ewp)
    y3 = _dense3(p2[0, :_N], p2[1, :_N], y2, dis, W3.reshape(1, _D), b2.reshape(1, _D))

    q = _agg_narrow(y3[:, 0], srcp, dstp, ewp)
    out = _final(q[0][:, None], q[1][:, None], y3, dis, b3.reshape(1, 1))
    return out[:, 0]


# speedup vs baseline: 11.8946x; 1.0025x over previous
import functools

import jax
import jax.numpy as jnp
from jax import lax
from jax.experimental import pallas as pl
from jax.experimental.pallas import tpu as pltpu
from jax.experimental.pallas import tpu_sc as plsc

_N = 10000
_D = 128
_E = 320000
_NC = 2
_NS = 16
_L = 16
_NW = _NC * _NS
_CH = 128
_CPW = 80
_EP = _NW * _CPW * _CH
_NP = 10240
_RPT = _NP // _NS

_mesh = plsc.VectorSubcoreMesh(
    core_axis_name="c", subcore_axis_name="s",
    num_cores=_NC, num_subcores=_NS)


@functools.partial(
    pl.kernel,
    compiler_params=pltpu.CompilerParams(needs_layout_passes=False),
    out_type=jax.ShapeDtypeStruct((_NC, _N), jnp.float32),
    mesh=_mesh,
    scratch_types=[
        pltpu.VMEM((_CPW, _CH), jnp.int32),
        pltpu.VMEM((_CPW, _CH), jnp.int32),
        pltpu.VMEM((_CPW, _CH), jnp.float32),
        pltpu.VMEM((_CPW, _CH), jnp.float32),
        pltpu.VMEM((_N,), jnp.float32),
        pltpu.VMEM((_N,), jnp.float32),
        pltpu.VMEM_SHARED((_N,), jnp.float32),
    ],
)
def _agg_narrow(y_hbm, src_hbm, dst_hbm, ew_hbm, out_hbm,
                src_v, dst_v, ew_v, msg_v, y_v, stage_v, acc_sh):
    c = lax.axis_index("c")
    s = lax.axis_index("s")
    w = c * _NS + s
    pltpu.sync_copy(src_hbm.at[w], src_v)
    pltpu.sync_copy(dst_hbm.at[w], dst_v)
    pltpu.sync_copy(ew_hbm.at[w], ew_v)
    pltpu.sync_copy(y_hbm, y_v)

    @pl.when(s == 0)
    def _():
        def zero_body(i, carry):
            stage_v[pl.ds(i * _L, _L)] = jnp.zeros((_L,), jnp.float32)
            return carry
        lax.fori_loop(0, _N // _L, zero_body, 0)
        pltpu.sync_copy(stage_v, acc_sh)

    plsc.subcore_barrier()

    def chunk_body(j, carry):
        for g in range(_CH // _L):
            sl = pl.ds(g * _L, _L)
            vals = plsc.load_gather(y_v, [src_v[j, sl]])
            msg_v[j, sl] = vals * ew_v[j, sl]
        pltpu.sync_copy(msg_v.at[j], acc_sh.at[dst_v.at[j]], add=True)
        return carry
    lax.fori_loop(0, _CPW, chunk_body, 0)

    plsc.subcore_barrier()

    @pl.when(s == 0)
    def _():
        pltpu.sync_copy(acc_sh, out_hbm.at[c])


_SCH = 32
_NSUB = _CH // _SCH
_TCH = _CPW * _NSUB
_NB = 4
_OUTER = _TCH // _NB


@functools.partial(
    pl.kernel,
    compiler_params=pltpu.CompilerParams(needs_layout_passes=False),
    out_type=jax.ShapeDtypeStruct((_NC, _NP, _D), jnp.float32),
    mesh=_mesh,
    scratch_types=[
        pltpu.VMEM((_CPW, _CH), jnp.int32),
        pltpu.VMEM((_CPW, _CH), jnp.int32),
        pltpu.VMEM((_CPW, _CH), jnp.float32),
        pltpu.VMEM((_SCH, _D), jnp.float32),
        pltpu.VMEM((_SCH, _D), jnp.float32),
        pltpu.VMEM((_SCH, _D), jnp.float32),
        pltpu.VMEM((_SCH, _D), jnp.float32),
        pltpu.VMEM_SHARED((_NP, _D), jnp.float32),
        pltpu.SemaphoreType.DMA,
        pltpu.SemaphoreType.DMA,
        pltpu.SemaphoreType.DMA,
        pltpu.SemaphoreType.DMA,
        pltpu.SemaphoreType.DMA,
        pltpu.SemaphoreType.DMA,
        pltpu.SemaphoreType.DMA,
        pltpu.SemaphoreType.DMA,
    ],
)
def _agg_wide(y_hbm, src_hbm, dst_hbm, ew_hbm, out_hbm,
              src_v, dst_v, ew_v, rows0, rows1, rows2, rows3,
              acc_sh, gs0, gs1, gs2, gs3, ss0, ss1, ss2, ss3):
    bufs = (rows0, rows1, rows2, rows3)
    gsems = (gs0, gs1, gs2, gs3)
    ssems = (ss0, ss1, ss2, ss3)
    c = lax.axis_index("c")
    s = lax.axis_index("s")
    w = c * _NS + s
    pltpu.sync_copy(src_hbm.at[w], src_v)
    pltpu.sync_copy(dst_hbm.at[w], dst_v)
    pltpu.sync_copy(ew_hbm.at[w], ew_v)

    def zero_body(i, carry):
        for f in range(_D // _L):
            rows0[i, pl.ds(f * _L, _L)] = jnp.zeros((_L,), jnp.float32)
        return carry
    lax.fori_loop(0, _SCH, zero_body, 0)
    for p in range(_RPT // _SCH):
        pltpu.async_copy(
            rows0, acc_sh.at[pl.ds(s * _RPT + p * _SCH, _SCH)],
            gsems[p % _NB])
    for p in range(_RPT // _SCH):
        pltpu.make_async_copy(
            rows0, acc_sh.at[pl.ds(s * _RPT + p * _SCH, _SCH)],
            gsems[p % _NB]).wait()

    plsc.subcore_barrier()

    def _gather_descr(t, buf, sem):
        j = t // _NSUB
        o = (t % _NSUB) * _SCH
        return pltpu.make_async_copy(
            y_hbm.at[src_v.at[j, pl.ds(o, _SCH)]], buf, sem)

    def _scatter_descr(t, buf, sem):
        j = t // _NSUB
        o = (t % _NSUB) * _SCH
        return pltpu.make_async_copy(
            buf, acc_sh.at[dst_v.at[j, pl.ds(o, _SCH)]], sem)

    for b in range(2):
        _gather_descr(b, bufs[b], gsems[b]).start()

    def outer_body(i, carry):
        for b in range(_NB):
            t = i * _NB + b
            j = t // _NSUB
            o = (t % _NSUB) * _SCH
            buf = bufs[b]
            nb = (b + 2) % _NB
            _gather_descr(t, buf, gsems[b]).wait()
            jv = jnp.full((_L,), j, jnp.int32)
            ov = jnp.full((_L,), o, jnp.int32)

            def edge_body(e, c2, buf=buf, jv=jv, ov=ov):
                scale = plsc.load_gather(
                    ew_v, [jv, ov + jnp.full((_L,), e, jnp.int32)])
                for f in range(_D // _L):
                    sl = pl.ds(f * _L, _L)
                    buf[e, sl] = buf[e, sl] * scale
                return c2
            lax.fori_loop(0, _SCH, edge_body, 0)
            _scatter_descr(t, buf, ssems[b]).start(add=True)

            if b < 2:
                @pl.when(i > 0)
                def _(t=t, nb=nb):
                    _scatter_descr(t - 2, bufs[nb], ssems[nb]).wait()
                _gather_descr(t + 2, bufs[nb], gsems[nb]).start()
            else:
                _scatter_descr(t - 2, bufs[nb], ssems[nb]).wait()

                @pl.when(i < _OUTER - 1)
                def _(t=t, nb=nb):
                    _gather_descr(t + 2, bufs[nb], gsems[nb]).start()
        return carry
    lax.fori_loop(0, _OUTER, outer_body, 0)

    for b in range(2, _NB):
        _scatter_descr(_TCH - _NB + b, bufs[b], ssems[b]).wait()

    plsc.subcore_barrier()

    pltpu.sync_copy(acc_sh.at[pl.ds(s * _RPT, _RPT)],
                    out_hbm.at[c, pl.ds(s * _RPT, _RPT)])


_R = 1000
_G = _N // _R


def _row_spec():
    return pl.BlockSpec((_R, _D), lambda i: (i, 0))


def _col_spec():
    return pl.BlockSpec((_R, 1), lambda i: (i, 0))


def _dense1_body(d0_ref, d1_ref, x_ref, w_ref, dis_ref, y_ref):
    deg = d0_ref[...] + d1_ref[...] + 1.0
    dis = lax.rsqrt(deg)
    xw = jnp.dot(x_ref[...], w_ref[...], preferred_element_type=jnp.float32)
    dis_ref[...] = dis
    y_ref[...] = dis * xw


def _dense1(d0, d1, x, W):
    return pl.pallas_call(
        _dense1_body,
        grid=(_G,),
        in_specs=[_col_spec(), _col_spec(), _row_spec(),
                  pl.BlockSpec((_D, _D), lambda i: (0, 0))],
        out_specs=[_col_spec(), _row_spec()],
        out_shape=[jax.ShapeDtypeStruct((_N, 1), jnp.float32),
                   jax.ShapeDtypeStruct((_N, _D), jnp.float32)],
    )(d0, d1, x, W)


def _dense_mid_body(p0_ref, p1_ref, y_ref, dis_ref, w_ref, b_ref, yn_ref):
    dis = dis_ref[...]
    h = jnp.maximum(
        dis * (p0_ref[...] + p1_ref[...] + y_ref[...]) + b_ref[...], 0.0)
    yn_ref[...] = dis * jnp.dot(h, w_ref[...],
                                preferred_element_type=jnp.float32)


def _dense_mid(p0, p1, y, dis, W, b):
    return pl.pallas_call(
        _dense_mid_body,
        grid=(_G,),
        in_specs=[_row_spec(), _row_spec(), _row_spec(), _col_spec(),
                  pl.BlockSpec((_D, _D), lambda i: (0, 0)),
                  pl.BlockSpec((1, _D), lambda i: (0, 0))],
        out_specs=_row_spec(),
        out_shape=jax.ShapeDtypeStruct((_N, _D), jnp.float32),
    )(p0, p1, y, dis, W, b)


def _dense3_body(p0_ref, p1_ref, y_ref, dis_ref, w3_ref, b_ref, y3_ref):
    dis = dis_ref[...]
    h = jnp.maximum(
        dis * (p0_ref[...] + p1_ref[...] + y_ref[...]) + b_ref[...], 0.0)
    y3_ref[...] = dis * jnp.sum(h * w3_ref[...], axis=1, keepdims=True)


def _dense3(p0, p1, y, dis, w3row, b):
    return pl.pallas_call(
        _dense3_body,
        grid=(_G,),
        in_specs=[_row_spec(), _row_spec(), _row_spec(), _col_spec(),
                  pl.BlockSpec((1, _D), lambda i: (0, 0)),
                  pl.BlockSpec((1, _D), lambda i: (0, 0))],
        out_specs=_col_spec(),
        out_shape=jax.ShapeDtypeStruct((_N, 1), jnp.float32),
    )(p0, p1, y, dis, w3row, b)


def _final_body(q0_ref, q1_ref, y3_ref, dis_ref, b3_ref, o_ref):
    o_ref[...] = (dis_ref[...] * (q0_ref[...] + q1_ref[...] + y3_ref[...])
                  + b3_ref[...])


def _final(q0, q1, y3, dis, b3):
    return pl.pallas_call(
        _final_body,
        grid=(_G,),
        in_specs=[_col_spec(), _col_spec(), _col_spec(), _col_spec(),
                  pl.BlockSpec((1, 1), lambda i: (0, 0))],
        out_specs=_col_spec(),
        out_shape=jax.ShapeDtypeStruct((_N, 1), jnp.float32),
    )(q0, q1, y3, dis, b3)


def kernel(x, edge_index, edge_attr, W1, b1, W2, b2, W3, b3):
    ei = edge_index.astype(jnp.int32)
    src, dst = ei[0], ei[1]
    ew = edge_attr[:, 0].astype(jnp.float32)

    pad = _EP - _E
    srcp = jnp.concatenate([src, jnp.zeros((pad,), jnp.int32)])
    dstp = jnp.concatenate([dst, jnp.zeros((pad,), jnp.int32)])
    ewp = jnp.concatenate([ew, jnp.zeros((pad,), jnp.float32)])
    srcp = srcp.reshape(_NW, _CPW, _CH)
    dstp = dstp.reshape(_NW, _CPW, _CH)
    ewp = ewp.reshape(_NW, _CPW, _CH)

    ones_n = jnp.ones((_N,), jnp.float32)
    degp = _agg_narrow(ones_n, srcp, dstp, ewp)
    dis, y1 = _dense1(degp[0][:, None], degp[1][:, None], x, W1)

    p1 = _agg_wide(y1, srcp, dstp, ewp)
    y2 = _dense_mid(p1[0, :_N], p1[1, :_N], y1, dis, W2, b1.reshape(1, _D))

    p2 = _agg_wide(y2, srcp, dstp, ewp)
    y3 = _dense3(p2[0, :_N], p2[1, :_N], y2, dis, W3.reshape(1, _D), b2.reshape(1, _D))

    q = _agg_narrow(y3[:, 0], srcp, dstp, ewp)
    out = _final(q[0][:, None], q[1][:, None], y3, dis, b3.reshape(1, 1))
    return out[:, 0]
